# baseline (speedup 1.0000x reference)
import functools

import jax
import jax.numpy as jnp
from jax import lax
from jax.experimental import pallas as pl
from jax.experimental.pallas import tpu as pltpu
from jax.experimental.pallas import tpu_sc as plsc

D = 64
SUB = 128
NC = 2
NS = 16
NW = NC * NS
NI = 16384
NJ = 50
IBLK = NI // SUB
BPW = IBLK // NW
GRP = 2
NU = NJ * (BPW // GRP)


@functools.partial(
    pl.kernel,
    out_type=jax.ShapeDtypeStruct((NJ, 8, IBLK, 8, SUB), jnp.float32),
    mesh=plsc.VectorSubcoreMesh(core_axis_name="c", subcore_axis_name="s"),
    compiler_params=pltpu.CompilerParams(use_tc_tiling_on_sc=False,
                                         needs_layout_passes=False),
    scratch_types=[
        pltpu.VMEM((NJ, BPW, SUB), jnp.int32),
        pltpu.VMEM((GRP * SUB, D), jnp.float32),
        pltpu.VMEM((GRP * SUB, D), jnp.float32),
        pltpu.VMEM((8, GRP, 8, SUB), jnp.float32),
        pltpu.VMEM((8, GRP, 8, SUB), jnp.float32),
        pltpu.SemaphoreType.DMA,
        pltpu.SemaphoreType.DMA,
        pltpu.SemaphoreType.DMA,
        pltpu.SemaphoreType.DMA,
    ],
)
def _emb_gather(idx_hbm, table_hbm, out_hbm, idx_v, rows0, rows1,
                trans0, trans1, gsem0, gsem1, wsem0, wsem1):
    cid = lax.axis_index("c")
    sid = lax.axis_index("s")
    wid = sid * NC + cid
    blk0 = wid * BPW

    rows = (rows0, rows1)
    trans = (trans0, trans1)
    gsems = (gsem0, gsem1)
    wsems = (wsem0, wsem1)
    iota16 = lax.iota(jnp.int32, 16)
    rowvs = [[iota16 + (blk * SUB + 16 * g) for g in range(8)]
             for blk in range(GRP)]

    def start_gathers(j, b):
        for blk in range(GRP):
            pltpu.async_copy(table_hbm.at[idx_v.at[j, GRP * b + blk]],
                             rows[b].at[pl.ds(blk * SUB, SUB)], gsems[b])

    def wait_gathers(b):
        for blk in range(GRP):
            pltpu.make_async_copy(table_hbm.at[idx_v.at[0, GRP * b + blk]],
                                  rows[b].at[pl.ds(blk * SUB, SUB)],
                                  gsems[b]).wait()

    def transpose(b):
        rows_b, trans_b = rows[b], trans[b]
        for blk in range(GRP):

            def body(d8, blk=blk):
                for dl in range(8):
                    colv = jnp.full((16,), dl, jnp.int32) + 8 * d8
                    for g in range(8):
                        vec = plsc.load_gather(rows_b, [rowvs[blk][g], colv])
                        trans_b[d8, blk, dl, pl.ds(16 * g, 16)] = vec

            plsc.parallel_loop(0, 8, unroll=2)(body)

    def start_writes(j, b):
        for d8 in range(8):
            pltpu.async_copy(trans[b].at[d8],
                             out_hbm.at[j, d8, pl.ds(blk0 + GRP * b, GRP)],
                             wsems[b])

    def wait_writes(b):
        for d8 in range(8):
            pltpu.make_async_copy(trans[b].at[d8],
                                  out_hbm.at[0, d8, pl.ds(blk0, GRP)],
                                  wsems[b]).wait()

    pltpu.sync_copy(idx_hbm.at[:, pl.ds(blk0, BPW)], idx_v)

    start_gathers(0, 0)

    def pair(k, carry):
        for b in (0, 1):
            u = 2 * k + b
            wait_gathers(b)

            @pl.when(u + 1 < NU)
            def _():
                start_gathers((u + 1) // 2, 1 - b)

            @pl.when(u >= 2)
            def _():
                wait_writes(b)

            transpose(b)
            start_writes(k, b)
        return carry

    lax.fori_loop(0, NU // 2, pair, 0)
    wait_writes(0)
    wait_writes(1)


def kernel(index, emb_weight):
    idx3 = jnp.transpose(index.astype(jnp.int32)).reshape(NJ, IBLK, SUB)
    out5 = _emb_gather(idx3, emb_weight)
    return out5.transpose(2, 4, 0, 1, 3).reshape(NI, NJ, D)


# speedup vs baseline: 1.8111x; 1.8111x over previous
import functools

import jax
import jax.numpy as jnp
from jax import lax
from jax.experimental import pallas as pl
from jax.experimental.pallas import tpu as pltpu
from jax.experimental.pallas import tpu_sc as plsc

D = 64
SUB = 128
NC = 2
NS = 16
NW = NC * NS
NI = 16384
NJ = 50
IBLK = NI // SUB
BPW = IBLK // NW
GRP = 2
NU = NJ * (BPW // GRP)


@functools.partial(
    pl.kernel,
    out_type=jax.ShapeDtypeStruct((NJ, 8, IBLK, 8, SUB), jnp.float32),
    mesh=plsc.VectorSubcoreMesh(core_axis_name="c", subcore_axis_name="s"),
    compiler_params=pltpu.CompilerParams(use_tc_tiling_on_sc=False,
                                         needs_layout_passes=False),
    scratch_types=[
        pltpu.VMEM((NJ, BPW, SUB), jnp.int32),
        pltpu.VMEM((GRP * SUB, D), jnp.float32),
        pltpu.VMEM((GRP * SUB, D), jnp.float32),
        pltpu.VMEM((8, GRP, 8, SUB + 1), jnp.float32),
        pltpu.VMEM((8, GRP, 8, SUB + 1), jnp.float32),
        pltpu.SemaphoreType.DMA,
        pltpu.SemaphoreType.DMA,
        pltpu.SemaphoreType.DMA,
        pltpu.SemaphoreType.DMA,
    ],
)
def _emb_gather(idx_hbm, table_hbm, out_hbm, idx_v, rows0, rows1,
                trans0, trans1, gsem0, gsem1, wsem0, wsem1):
    cid = lax.axis_index("c")
    sid = lax.axis_index("s")
    wid = sid * NC + cid
    blk0 = wid * BPW

    rows = (rows0, rows1)
    trans = (trans0, trans1)
    gsems = (gsem0, gsem1)
    wsems = (wsem0, wsem1)
    iota16 = lax.iota(jnp.int32, 16)
    rowvs = [[iota16 + (blk * SUB + 16 * g) for g in range(8)]
             for blk in range(GRP)]

    def start_gathers(j, b):
        for blk in range(GRP):
            pltpu.async_copy(table_hbm.at[idx_v.at[j, GRP * b + blk]],
                             rows[b].at[pl.ds(blk * SUB, SUB)], gsems[b])

    def wait_gathers(b):
        for blk in range(GRP):
            pltpu.make_async_copy(table_hbm.at[idx_v.at[0, GRP * b + blk]],
                                  rows[b].at[pl.ds(blk * SUB, SUB)],
                                  gsems[b]).wait()

    d8vs = [iota16 // 8 + 2 * q for q in range(4)]
    dlv = iota16 % 8

    def transpose(b):
        rows_b, trans_b = rows[b], trans[b]
        for blk in range(GRP):
            blkv = jnp.full((16,), blk, jnp.int32)

            def body(il, blk=blk, blkv=blkv):
                ilv = jnp.full((16,), 0, jnp.int32) + il
                for q in range(4):
                    vec = rows_b[blk * SUB + il, pl.ds(16 * q, 16)]
                    plsc.store_scatter(trans_b, [d8vs[q], blkv, dlv, ilv],
                                       vec)

            plsc.parallel_loop(0, SUB)(body)

    def start_writes(j, b):
        for d8 in range(8):
            for blk in range(GRP):
                pltpu.async_copy(
                    trans[b].at[d8, blk, pl.ds(0, 8), pl.ds(0, SUB)],
                    out_hbm.at[j, d8, blk0 + GRP * b + blk], wsems[b])

    def wait_writes(b):
        for d8 in range(8):
            for blk in range(GRP):
                pltpu.make_async_copy(
                    trans[b].at[d8, blk, pl.ds(0, 8), pl.ds(0, SUB)],
                    out_hbm.at[0, d8, blk0], wsems[b]).wait()

    pltpu.sync_copy(idx_hbm.at[:, pl.ds(blk0, BPW)], idx_v)

    start_gathers(0, 0)

    def pair(k, carry):
        for b in (0, 1):
            u = 2 * k + b
            wait_gathers(b)

            @pl.when(u + 1 < NU)
            def _():
                start_gathers((u + 1) // 2, 1 - b)

            @pl.when(u >= 2)
            def _():
                wait_writes(b)

            transpose(b)
            start_writes(k, b)
        return carry

    lax.fori_loop(0, NU // 2, pair, 0)
    wait_writes(0)
    wait_writes(1)


def kernel(index, emb_weight):
    idx3 = jnp.transpose(index.astype(jnp.int32)).reshape(NJ, IBLK, SUB)
    out5 = _emb_gather(idx3, emb_weight)
    return out5.transpose(2, 4, 0, 1, 3).reshape(NI, NJ, D)
